# baseline (speedup 1.0000x reference)
import functools

import jax
import jax.numpy as jnp
from jax import lax
from jax.experimental import pallas as pl
from jax.experimental.pallas import tpu as pltpu, tpu_sc as plsc

N_NODES = 100000
MAXDEG = 64
SAMPLES1 = 25
SAMPLES2 = 10
BATCH = 4096
B1 = BATCH * SAMPLES2

_PERM0 = [17, 27, 42, 32, 1, 3, 58, 51, 40, 28]
_PERM1 = [2, 32, 15, 10, 48, 25, 28, 0, 49, 4, 60, 42, 21, 11, 20,
          57, 17, 12, 19, 22, 18, 16, 27, 5, 23]

_MESH = plsc.VectorSubcoreMesh(
    core_axis_name="c", subcore_axis_name="s", num_cores=2, num_subcores=16)
_PARAMS = pltpu.CompilerParams(
    needs_layout_passes=False, use_tc_tiling_on_sc=True)

N_SLABS = 8
SLAB = B1 // N_SLABS


@functools.partial(
    pl.kernel,
    out_type=(
        jax.ShapeDtypeStruct((SAMPLES2, BATCH), jnp.int32),
        jax.ShapeDtypeStruct((B1,), jnp.int32),
    ),
    mesh=_MESH,
    compiler_params=_PARAMS,
    scratch_types=[
        pltpu.VMEM((N_NODES,), jnp.int32),
        pltpu.VMEM((BATCH,), jnp.int32),
        pltpu.VMEM((1, BATCH), jnp.int32),
        pltpu.VMEM((BATCH,), jnp.int32),
        pltpu.SemaphoreType.DMA,
        pltpu.SemaphoreType.DMA,
    ],
)
def _hop0_kernel(adjt_h, x_h, out0_h, idxf_h, col_v, xs_v, val_v, pos_v,
                 sem, sem2):
  wid = lax.axis_index("s") * 2 + lax.axis_index("c")
  lane = lax.iota(jnp.int32, 16)
  src_col = jnp.int32(_PERM0[-1])
  for j in range(SAMPLES2 - 1):
    src_col = jnp.where(wid == j, jnp.int32(_PERM0[j]), src_col)

  @pl.when(wid < SAMPLES2)
  def _():
    pltpu.sync_copy(adjt_h.at[pl.ds(src_col, 1), :].at[0], col_v)
    pltpu.sync_copy(x_h, xs_v)

    def step(t, carry):
      base = t * 16
      iv = xs_v[pl.ds(base, 16)]
      val_v[0, pl.ds(base, 16)] = plsc.load_gather(col_v, [iv])
      pos_v[pl.ds(base, 16)] = (base + lane) * SAMPLES2 + wid
      return carry

    lax.fori_loop(0, BATCH // 16, step, 0)
    pltpu.async_copy(val_v, out0_h.at[pl.ds(wid, 1), :], sem).wait()
    pltpu.async_copy(val_v.at[0], idxf_h.at[pos_v], sem2).wait()


@functools.partial(
    pl.kernel,
    out_type=jax.ShapeDtypeStruct((SAMPLES1, B1), jnp.int32),
    mesh=_MESH,
    compiler_params=_PARAMS,
    scratch_types=[
        pltpu.VMEM((N_NODES,), jnp.int32),
        pltpu.VMEM((SLAB,), jnp.int32),
        pltpu.VMEM((SLAB,), jnp.int32),
        pltpu.VMEM((1, SLAB), jnp.int32),
        pltpu.VMEM((1, SLAB), jnp.int32),
        pltpu.SemaphoreType.DMA,
        pltpu.SemaphoreType.DMA,
        pltpu.SemaphoreType.DMA,
        pltpu.SemaphoreType.DMA,
        pltpu.SemaphoreType.DMA,
    ],
)
def _hop1_kernel(adjt_h, idxf_h, out1_h, col_v, idx0_v, idx1_v,
                 stg0_v, stg1_v, csem, isem0, isem1, osem0, osem1):
  wid = lax.axis_index("s") * 2 + lax.axis_index("c")
  idxs = [idx0_v, idx1_v]
  stgs = [stg0_v, stg1_v]
  isems = [isem0, isem1]
  osems = [osem0, osem1]
  src_col = jnp.int32(_PERM1[-1])
  for j in range(SAMPLES1 - 1):
    src_col = jnp.where(wid == j, jnp.int32(_PERM1[j]), src_col)

  @pl.when(wid < SAMPLES1)
  def _():
    ccol = pltpu.async_copy(
        adjt_h.at[pl.ds(src_col, 1), :].at[0], col_v, csem)
    in_copies = [None, None]
    out_copies = [None, None]
    in_copies[0] = pltpu.async_copy(
        idxf_h.at[pl.ds(0, SLAB)], idx0_v, isem0)
    ccol.wait()
    for s in range(N_SLABS):
      b = s % 2
      if s + 1 < N_SLABS:
        in_copies[(s + 1) % 2] = pltpu.async_copy(
            idxf_h.at[pl.ds((s + 1) * SLAB, SLAB)], idxs[(s + 1) % 2],
            isems[(s + 1) % 2])
      in_copies[b].wait()
      if out_copies[b] is not None:
        out_copies[b].wait()
      idx_b, stg_b = idxs[b], stgs[b]

      def step(t, carry):
        for u in range(4):
          base = (t * 4 + u) * 16
          iv = idx_b[pl.ds(base, 16)]
          stg_b[0, pl.ds(base, 16)] = plsc.load_gather(col_v, [iv])
        return carry

      lax.fori_loop(0, SLAB // 64, step, 0)
      out_copies[b] = pltpu.async_copy(
          stg_b, out1_h.at[pl.ds(wid, 1), pl.ds(s * SLAB, SLAB)], osems[b])
    out_copies[0].wait()
    out_copies[1].wait()


def kernel(x, adj_table):
  adj_t = adj_table.T
  out0t, idxf = _hop0_kernel(adj_t, x)
  out1t = _hop1_kernel(adj_t, idxf)
  return (out0t.T, out1t.T)


# speedup vs baseline: 2.3980x; 2.3980x over previous
import functools

import jax
import jax.numpy as jnp
from jax import lax
from jax.experimental import pallas as pl
from jax.experimental.pallas import tpu as pltpu, tpu_sc as plsc

N_NODES = 100000
MAXDEG = 64
SAMPLES1 = 25
SAMPLES2 = 10
BATCH = 4096
B1 = BATCH * SAMPLES2

_PERM0 = [17, 27, 42, 32, 1, 3, 58, 51, 40, 28]
_PERM1 = [2, 32, 15, 10, 48, 25, 28, 0, 49, 4, 60, 42, 21, 11, 20,
          57, 17, 12, 19, 22, 18, 16, 27, 5, 23]

_MESH = plsc.VectorSubcoreMesh(
    core_axis_name="c", subcore_axis_name="s", num_cores=2, num_subcores=16)
_PARAMS = pltpu.CompilerParams(
    needs_layout_passes=False, use_tc_tiling_on_sc=True)

N_SLABS = 8
SLAB = B1 // N_SLABS
SCOLS = SLAB // SAMPLES2


@functools.partial(
    pl.kernel,
    out_type=jax.ShapeDtypeStruct((SAMPLES2, BATCH), jnp.int32),
    mesh=_MESH,
    compiler_params=_PARAMS,
    scratch_types=[
        pltpu.VMEM((N_NODES,), jnp.int32),
        pltpu.VMEM((BATCH,), jnp.int32),
        pltpu.VMEM((1, BATCH), jnp.int32),
        pltpu.SemaphoreType.DMA,
    ],
)
def _hop0_kernel(adjt_h, x_h, out0_h, col_v, xs_v, val_v, sem):
  wid = lax.axis_index("s") * 2 + lax.axis_index("c")
  src_col = jnp.int32(_PERM0[-1])
  for j in range(SAMPLES2 - 1):
    src_col = jnp.where(wid == j, jnp.int32(_PERM0[j]), src_col)

  @pl.when(wid < SAMPLES2)
  def _():
    pltpu.sync_copy(adjt_h.at[pl.ds(src_col, 1), :].at[0], col_v)
    pltpu.sync_copy(x_h, xs_v)

    def step(t, carry):
      for u in range(4):
        base = t * 64 + u * 16
        iv = xs_v[pl.ds(base, 16)]
        val_v[0, pl.ds(base, 16)] = plsc.load_gather(col_v, [iv])
      return carry

    lax.fori_loop(0, BATCH // 64, step, 0)
    pltpu.async_copy(val_v, out0_h.at[pl.ds(wid, 1), :], sem).wait()


@functools.partial(
    pl.kernel,
    out_type=jax.ShapeDtypeStruct((SAMPLES1, B1), jnp.int32),
    mesh=_MESH,
    compiler_params=_PARAMS,
    scratch_types=[
        pltpu.VMEM((N_NODES,), jnp.int32),
        pltpu.VMEM((SAMPLES2, SCOLS), jnp.int32),
        pltpu.VMEM((SAMPLES2, SCOLS), jnp.int32),
        pltpu.VMEM((1, SLAB), jnp.int32),
        pltpu.VMEM((1, SLAB), jnp.int32),
        pltpu.SemaphoreType.DMA,
        pltpu.SemaphoreType.DMA,
        pltpu.SemaphoreType.DMA,
        pltpu.SemaphoreType.DMA,
        pltpu.SemaphoreType.DMA,
    ],
)
def _hop1_kernel(adjt_h, hop0t_h, out1_h, col_v, idx0_v, idx1_v,
                 stg0_v, stg1_v, csem, isem0, isem1, osem0, osem1):
  wid = lax.axis_index("s") * 2 + lax.axis_index("c")
  lane = lax.iota(jnp.int32, 16)
  idxs = [idx0_v, idx1_v]
  stgs = [stg0_v, stg1_v]
  isems = [isem0, isem1]
  osems = [osem0, osem1]
  src_col = jnp.int32(_PERM1[-1])
  for j in range(SAMPLES1 - 1):
    src_col = jnp.where(wid == j, jnp.int32(_PERM1[j]), src_col)

  @pl.when(wid < SAMPLES1)
  def _():
    ccol = pltpu.async_copy(
        adjt_h.at[pl.ds(src_col, 1), :].at[0], col_v, csem)
    in_copies = [None, None]
    out_copies = [None, None]
    in_copies[0] = pltpu.async_copy(
        hop0t_h.at[:, pl.ds(0, SCOLS)], idx0_v, isem0)
    ccol.wait()
    for s in range(N_SLABS):
      b = s % 2
      if s + 1 < N_SLABS:
        in_copies[(s + 1) % 2] = pltpu.async_copy(
            hop0t_h.at[:, pl.ds((s + 1) * SCOLS, SCOLS)], idxs[(s + 1) % 2],
            isems[(s + 1) % 2])
      in_copies[b].wait()
      if out_copies[b] is not None:
        out_copies[b].wait()
      idx_b, stg_b = idxs[b], stgs[b]

      jv0 = lane - 10 * (lane >= 10).astype(jnp.int32)
      dv0 = (lane >= 10).astype(jnp.int32)

      def step(t, carry):
        jv, dv = carry
        base = t * 16
        iv = plsc.load_gather(idx_b, [jv, dv])
        stg_b[0, pl.ds(base, 16)] = plsc.load_gather(col_v, [iv])
        jn = jv + 6
        wrap = (jn >= 10).astype(jnp.int32)
        return (jn - 10 * wrap, dv + 1 + wrap)

      lax.fori_loop(0, SLAB // 16, step, (jv0, dv0))
      out_copies[b] = pltpu.async_copy(
          stg_b, out1_h.at[pl.ds(wid, 1), pl.ds(s * SLAB, SLAB)], osems[b])
    out_copies[0].wait()
    out_copies[1].wait()


def kernel(x, adj_table):
  adj_t = adj_table.T
  out0t = _hop0_kernel(adj_t, x)
  out1t = _hop1_kernel(adj_t, out0t)
  return (out0t.T, out1t.T)
